# baseline (device time: 29624 ns/iter reference)
import jax
import jax.numpy as jnp
from jax import lax
from jax.experimental import pallas as pl
from jax.experimental.pallas import tpu as pltpu

N_DEV = 4
N_LOCAL_E = 4
N_CHUNK = 2
CAP = 128
QSCALE = 2.0 / 127.0


def kernel(x, router_W, route_idx, expert_W, shared_W):
    rows, d_model = x.shape
    d_ff = expert_W.shape[2]
    blk = rows // N_DEV
    half = blk // N_CHUNK
    n_slots = (N_DEV - 1) * N_CHUNK

    def body(x_hbm, rw_ref, idx_ref, ew_ref, sw_ref, out_ref,
             x_ref, send_buf, comm_buf, ew_vmem, sw_vmem, wcat16, sw16,
             w_sems, send_sems, recv_sems):
        my = lax.axis_index("i")

        x_dma = pltpu.make_async_copy(x_hbm, x_ref, w_sems.at[2])
        x_dma.start()
        ew_dma = pltpu.make_async_copy(ew_ref, ew_vmem, w_sems.at[0])
        sw_dma = pltpu.make_async_copy(sw_ref, sw_vmem, w_sems.at[1])
        ew_dma.start()
        sw_dma.start()

        barrier = pltpu.get_barrier_semaphore()
        for p in range(1, N_DEV):
            pl.semaphore_signal(
                barrier, inc=1,
                device_id=((my + p) % N_DEV,),
                device_id_type=pl.DeviceIdType.MESH,
            )
        pl.semaphore_wait(barrier, N_DEV - 1)

        ltri = (
            lax.broadcasted_iota(jnp.int32, (half, half), 1)
            < lax.broadcasted_iota(jnp.int32, (half, half), 0)
        ).astype(jnp.bfloat16)

        def pack_matrix(c_sl, owner):
            route = idx_ref[c_sl, :]
            match = (route // N_LOCAL_E) == owner
            m16 = match.astype(jnp.bfloat16)
            pos = jnp.dot(ltri, m16, preferred_element_type=jnp.float32)
            pt = (
                lax.broadcasted_iota(jnp.int32, (half, CAP), 1)
                == pos.astype(jnp.int32)
            ) & match
            return pt.astype(jnp.bfloat16), route

        def gather_xcat(c_sl, pt, route):
            xb16 = x_ref[c_sl, :].astype(jnp.bfloat16)
            xg = lax.dot_general(
                pt, xb16, (((0,), (0,)), ((), ())),
                preferred_element_type=jnp.float32,
            ).astype(jnp.bfloat16)
            rg = lax.dot_general(
                pt, route.astype(jnp.bfloat16), (((0,), (0,)), ((), ())),
                preferred_element_type=jnp.float32,
            )
            parts = [
                jnp.where(rg == (my * N_LOCAL_E + j), xg, jnp.zeros((), jnp.bfloat16))
                for j in range(N_LOCAL_E)
            ]
            return jnp.concatenate(parts, axis=1)

        def chunk_gate(c_sl):
            xb = x_ref[c_sl, :]
            scores = jnp.dot(xb, rw_ref[:, :], preferred_element_type=jnp.float32)
            scores = scores - jnp.max(scores, axis=1, keepdims=True)
            probs = jnp.exp(scores)
            probs = probs / jnp.sum(probs, axis=1, keepdims=True)
            route = idx_ref[c_sl, :]
            onehot = lax.broadcasted_iota(jnp.int32, probs.shape, 1) == route
            return jnp.sum(jnp.where(onehot, probs, 0.0), axis=1, keepdims=True)

        recv_pts = []
        for c in range(N_CHUNK):
            c_sl = pl.ds(my * blk + c * half, half)
            for s in range(1, N_DEV):
                src = (my - s) % N_DEV
                recv_pts.append(pack_matrix(c_sl, src)[0])

        x_dma.wait()
        xcats = []
        for s in range(1, N_DEV):
            dest = (my + s) % N_DEV
            for c in range(N_CHUNK):
                c_sl = pl.ds(dest * blk + c * half, half)
                pt, route = pack_matrix(c_sl, my)
                xcats.append(gather_xcat(c_sl, pt, route))

        own_pre = []
        for c in range(N_CHUNK):
            c_sl = pl.ds(my * blk + c * half, half)
            gate = chunk_gate(c_sl)
            pt, route = pack_matrix(c_sl, my)
            own_pre.append((gate, pt, gather_xcat(c_sl, pt, route)))

        ew_dma.wait()
        sw_dma.wait()
        wcat16[:, :] = jnp.reshape(
            ew_vmem[:, :, :], (N_LOCAL_E * d_model, d_ff)
        ).astype(jnp.bfloat16)
        sw16[:, :] = sw_vmem[:, :].astype(jnp.bfloat16)

        rdmas = []
        for s in range(1, N_DEV):
            dest = (my + s) % N_DEV
            for c in range(N_CHUNK):
                slot = (s - 1) * N_CHUNK + c
                v = jnp.dot(xcats[slot], wcat16[:, :],
                            preferred_element_type=jnp.float32)
                q = jnp.clip(v * (1.0 / QSCALE), -127.0, 127.0)
                send_buf[slot, :, :] = jnp.round(q).astype(jnp.int8)
                rdma = pltpu.make_async_remote_copy(
                    src_ref=send_buf.at[slot],
                    dst_ref=comm_buf.at[slot],
                    send_sem=send_sems.at[slot],
                    recv_sem=recv_sems.at[slot],
                    device_id=(dest,),
                    device_id_type=pl.DeviceIdType.MESH,
                )
                rdma.start()
                rdmas.append(rdma)

        gates = []
        for c in range(N_CHUNK):
            gate, pt, xcat = own_pre[c]
            gates.append(gate)
            c_sl = pl.ds(my * blk + c * half, half)
            v = jnp.dot(xcat, wcat16[:, :], preferred_element_type=jnp.float32)
            own = jnp.dot(pt, v.astype(jnp.bfloat16),
                          preferred_element_type=jnp.float32) * gate
            xb16 = x_ref[c_sl, :].astype(jnp.bfloat16)
            own = own + jnp.dot(xb16, sw16[:, :], preferred_element_type=jnp.float32)
            out_ref[pl.ds(c * half, half), :] = own

        for c in range(N_CHUNK):
            for s in range(1, N_DEV):
                rdmas[(s - 1) * N_CHUNK + c].wait_recv()
            acc = jnp.zeros((half, d_ff), jnp.float32)
            for s in range(1, N_DEV):
                pt = recv_pts[c * (N_DEV - 1) + (s - 1)]
                q16 = comm_buf[(s - 1) * N_CHUNK + c, :, :].astype(jnp.bfloat16)
                acc = acc + jnp.dot(pt, q16, preferred_element_type=jnp.float32)
            out_sl = pl.ds(c * half, half)
            out_ref[out_sl, :] = out_ref[out_sl, :] + acc * (gates[c] * QSCALE)

        for r in rdmas:
            r.wait_send()

    return pl.pallas_call(
        body,
        out_shape=jax.ShapeDtypeStruct((blk, d_ff), jnp.float32),
        in_specs=[
            pl.BlockSpec(memory_space=pl.ANY),
            pl.BlockSpec(memory_space=pltpu.VMEM),
            pl.BlockSpec(memory_space=pltpu.VMEM),
            pl.BlockSpec(memory_space=pl.ANY),
            pl.BlockSpec(memory_space=pl.ANY),
        ],
        out_specs=pl.BlockSpec(memory_space=pltpu.VMEM),
        scratch_shapes=[
            pltpu.VMEM((rows, d_model), jnp.float32),
            pltpu.VMEM((n_slots, CAP, d_ff), jnp.int8),
            pltpu.VMEM((n_slots, CAP, d_ff), jnp.int8),
            pltpu.VMEM((N_LOCAL_E, d_model, d_ff), jnp.float32),
            pltpu.VMEM((d_model, d_ff), jnp.float32),
            pltpu.VMEM((N_LOCAL_E * d_model, d_ff), jnp.bfloat16),
            pltpu.VMEM((d_model, d_ff), jnp.bfloat16),
            pltpu.SemaphoreType.DMA((3,)),
            pltpu.SemaphoreType.DMA((n_slots,)),
            pltpu.SemaphoreType.DMA((n_slots,)),
        ],
        compiler_params=pltpu.CompilerParams(collective_id=0),
    )(x, router_W, route_idx, expert_W, shared_W)


# device time: 29050 ns/iter; 1.0198x vs baseline; 1.0198x over previous
import jax
import jax.numpy as jnp
from jax import lax
from jax.experimental import pallas as pl
from jax.experimental.pallas import tpu as pltpu

N_DEV = 4
N_LOCAL_E = 4
N_CHUNK = 2
CAP = 128
QSCALE = 2.0 / 127.0


def kernel(x, router_W, route_idx, expert_W, shared_W):
    rows, d_model = x.shape
    d_ff = expert_W.shape[2]
    blk = rows // N_DEV
    half = blk // N_CHUNK
    n_slots = (N_DEV - 1) * N_CHUNK

    def body(x_ref, rw_ref, idx_ref, ew_ref, sw_ref, out_ref,
             send_buf, comm_buf, ew_vmem, sw_vmem, wcat16, sw16,
             w_sems, send_sems, recv_sems):
        my = lax.axis_index("i")

        ew_dma = pltpu.make_async_copy(ew_ref, ew_vmem, w_sems.at[0])
        sw_dma = pltpu.make_async_copy(sw_ref, sw_vmem, w_sems.at[1])
        ew_dma.start()
        sw_dma.start()

        barrier = pltpu.get_barrier_semaphore()
        for p in range(1, N_DEV):
            pl.semaphore_signal(
                barrier, inc=1,
                device_id=((my + p) % N_DEV,),
                device_id_type=pl.DeviceIdType.MESH,
            )
        pl.semaphore_wait(barrier, N_DEV - 1)

        ltri = (
            lax.broadcasted_iota(jnp.int32, (half, half), 1)
            < lax.broadcasted_iota(jnp.int32, (half, half), 0)
        ).astype(jnp.bfloat16)

        def pack_matrix(c_sl, owner):
            route = idx_ref[c_sl, :]
            match = (route // N_LOCAL_E) == owner
            m16 = match.astype(jnp.bfloat16)
            pos = jnp.dot(ltri, m16, preferred_element_type=jnp.float32)
            pt = (
                lax.broadcasted_iota(jnp.int32, (half, CAP), 1)
                == pos.astype(jnp.int32)
            ) & match
            return pt.astype(jnp.bfloat16), route

        def gather_xcat(c_sl, pt, route):
            xb16 = x_ref[c_sl, :].astype(jnp.bfloat16)
            xg = lax.dot_general(
                pt, xb16, (((0,), (0,)), ((), ())),
                preferred_element_type=jnp.float32,
            ).astype(jnp.bfloat16)
            rg = lax.dot_general(
                pt, route.astype(jnp.bfloat16), (((0,), (0,)), ((), ())),
                preferred_element_type=jnp.float32,
            )
            parts = [
                jnp.where(rg == (my * N_LOCAL_E + j), xg, jnp.zeros((), jnp.bfloat16))
                for j in range(N_LOCAL_E)
            ]
            return jnp.concatenate(parts, axis=1)

        def chunk_gate(c_sl):
            xb = x_ref[c_sl, :]
            scores = jnp.dot(xb, rw_ref[:, :], preferred_element_type=jnp.float32)
            scores = scores - jnp.max(scores, axis=1, keepdims=True)
            probs = jnp.exp(scores)
            probs = probs / jnp.sum(probs, axis=1, keepdims=True)
            route = idx_ref[c_sl, :]
            onehot = lax.broadcasted_iota(jnp.int32, probs.shape, 1) == route
            return jnp.sum(jnp.where(onehot, probs, 0.0), axis=1, keepdims=True)

        recv_pts = []
        for c in range(N_CHUNK):
            c_sl = pl.ds(my * blk + c * half, half)
            for s in range(1, N_DEV):
                src = (my - s) % N_DEV
                recv_pts.append(pack_matrix(c_sl, src)[0])

        xcats = []
        for s in range(1, N_DEV):
            dest = (my + s) % N_DEV
            for c in range(N_CHUNK):
                c_sl = pl.ds(dest * blk + c * half, half)
                pt, route = pack_matrix(c_sl, my)
                xcats.append(gather_xcat(c_sl, pt, route))

        own_pre = []
        for c in range(N_CHUNK):
            c_sl = pl.ds(my * blk + c * half, half)
            gate = chunk_gate(c_sl)
            pt, route = pack_matrix(c_sl, my)
            own_pre.append((gate, pt, gather_xcat(c_sl, pt, route)))

        ew_dma.wait()
        sw_dma.wait()
        wcat16[:, :] = jnp.reshape(
            ew_vmem[:, :, :], (N_LOCAL_E * d_model, d_ff)
        ).astype(jnp.bfloat16)
        sw16[:, :] = sw_vmem[:, :].astype(jnp.bfloat16)

        rdmas = []
        for s in range(1, N_DEV):
            dest = (my + s) % N_DEV
            for c in range(N_CHUNK):
                slot = (s - 1) * N_CHUNK + c
                v = jnp.dot(xcats[slot], wcat16[:, :],
                            preferred_element_type=jnp.float32)
                q = jnp.clip(v * (1.0 / QSCALE), -127.0, 127.0)
                send_buf[slot, :, :] = jnp.round(q).astype(jnp.int8)
                rdma = pltpu.make_async_remote_copy(
                    src_ref=send_buf.at[slot],
                    dst_ref=comm_buf.at[slot],
                    send_sem=send_sems.at[slot],
                    recv_sem=recv_sems.at[slot],
                    device_id=(dest,),
                    device_id_type=pl.DeviceIdType.MESH,
                )
                rdma.start()
                rdmas.append(rdma)

        gates = []
        for c in range(N_CHUNK):
            gate, pt, xcat = own_pre[c]
            gates.append(gate)
            c_sl = pl.ds(my * blk + c * half, half)
            v = jnp.dot(xcat, wcat16[:, :], preferred_element_type=jnp.float32)
            own = jnp.dot(pt, v.astype(jnp.bfloat16),
                          preferred_element_type=jnp.float32) * gate
            xb16 = x_ref[c_sl, :].astype(jnp.bfloat16)
            own = own + jnp.dot(xb16, sw16[:, :], preferred_element_type=jnp.float32)
            out_ref[pl.ds(c * half, half), :] = own

        for c in range(N_CHUNK):
            for s in range(1, N_DEV):
                rdmas[(s - 1) * N_CHUNK + c].wait_recv()
            acc = jnp.zeros((half, d_ff), jnp.float32)
            for s in range(1, N_DEV):
                pt = recv_pts[c * (N_DEV - 1) + (s - 1)]
                q16 = comm_buf[(s - 1) * N_CHUNK + c, :, :].astype(jnp.bfloat16)
                acc = acc + jnp.dot(pt, q16, preferred_element_type=jnp.float32)
            out_sl = pl.ds(c * half, half)
            out_ref[out_sl, :] = out_ref[out_sl, :] + acc * (gates[c] * QSCALE)

        for r in rdmas:
            r.wait_send()

    return pl.pallas_call(
        body,
        out_shape=jax.ShapeDtypeStruct((blk, d_ff), jnp.float32),
        in_specs=[
            pl.BlockSpec(memory_space=pltpu.VMEM),
            pl.BlockSpec(memory_space=pltpu.VMEM),
            pl.BlockSpec(memory_space=pltpu.VMEM),
            pl.BlockSpec(memory_space=pl.ANY),
            pl.BlockSpec(memory_space=pl.ANY),
        ],
        out_specs=pl.BlockSpec(memory_space=pltpu.VMEM),
        scratch_shapes=[
            pltpu.VMEM((n_slots, CAP, d_ff), jnp.int8),
            pltpu.VMEM((n_slots, CAP, d_ff), jnp.int8),
            pltpu.VMEM((N_LOCAL_E, d_model, d_ff), jnp.float32),
            pltpu.VMEM((d_model, d_ff), jnp.float32),
            pltpu.VMEM((N_LOCAL_E * d_model, d_ff), jnp.bfloat16),
            pltpu.VMEM((d_model, d_ff), jnp.bfloat16),
            pltpu.SemaphoreType.DMA((3,)),
            pltpu.SemaphoreType.DMA((n_slots,)),
            pltpu.SemaphoreType.DMA((n_slots,)),
        ],
        compiler_params=pltpu.CompilerParams(collective_id=0),
    )(x, router_W, route_idx, expert_W, shared_W)


# device time: 27499 ns/iter; 1.0773x vs baseline; 1.0564x over previous
import jax
import jax.numpy as jnp
from jax import lax
from jax.experimental import pallas as pl
from jax.experimental.pallas import tpu as pltpu

N_DEV = 4
N_LOCAL_E = 4
N_CHUNK = 1
CAP = 192
QSCALE = 2.0 / 127.0


def kernel(x, router_W, route_idx, expert_W, shared_W):
    rows, d_model = x.shape
    d_ff = expert_W.shape[2]
    blk = rows // N_DEV
    half = blk // N_CHUNK
    n_slots = (N_DEV - 1) * N_CHUNK

    def body(x_ref, rw_ref, idx_ref, ew_ref, sw_ref, out_ref,
             send_buf, comm_buf, ew_vmem, sw_vmem, wcat16, sw16,
             w_sems, send_sems, recv_sems):
        my = lax.axis_index("i")

        ew_dma = pltpu.make_async_copy(ew_ref, ew_vmem, w_sems.at[0])
        sw_dma = pltpu.make_async_copy(sw_ref, sw_vmem, w_sems.at[1])
        ew_dma.start()
        sw_dma.start()

        barrier = pltpu.get_barrier_semaphore()
        for p in range(1, N_DEV):
            pl.semaphore_signal(
                barrier, inc=1,
                device_id=((my + p) % N_DEV,),
                device_id_type=pl.DeviceIdType.MESH,
            )
        pl.semaphore_wait(barrier, N_DEV - 1)

        ltri = (
            lax.broadcasted_iota(jnp.int32, (half, half), 1)
            < lax.broadcasted_iota(jnp.int32, (half, half), 0)
        ).astype(jnp.bfloat16)

        def pack_matrix(c_sl, owner):
            route = idx_ref[c_sl, :]
            match = (route // N_LOCAL_E) == owner
            m16 = match.astype(jnp.bfloat16)
            pos = jnp.dot(ltri, m16, preferred_element_type=jnp.float32)
            pt = (
                lax.broadcasted_iota(jnp.int32, (half, CAP), 1)
                == pos.astype(jnp.int32)
            ) & match
            return pt.astype(jnp.bfloat16), route

        def gather_xcat(c_sl, pt, route):
            xb16 = x_ref[c_sl, :].astype(jnp.bfloat16)
            xg = lax.dot_general(
                pt, xb16, (((0,), (0,)), ((), ())),
                preferred_element_type=jnp.float32,
            ).astype(jnp.bfloat16)
            rg = lax.dot_general(
                pt, route.astype(jnp.bfloat16), (((0,), (0,)), ((), ())),
                preferred_element_type=jnp.float32,
            )
            parts = [
                jnp.where(rg == (my * N_LOCAL_E + j), xg, jnp.zeros((), jnp.bfloat16))
                for j in range(N_LOCAL_E)
            ]
            return jnp.concatenate(parts, axis=1)

        def chunk_gate(c_sl):
            xb = x_ref[c_sl, :]
            scores = jnp.dot(xb, rw_ref[:, :], preferred_element_type=jnp.float32)
            scores = scores - jnp.max(scores, axis=1, keepdims=True)
            probs = jnp.exp(scores)
            probs = probs / jnp.sum(probs, axis=1, keepdims=True)
            route = idx_ref[c_sl, :]
            onehot = lax.broadcasted_iota(jnp.int32, probs.shape, 1) == route
            return jnp.sum(jnp.where(onehot, probs, 0.0), axis=1, keepdims=True)

        recv_pts = []
        for c in range(N_CHUNK):
            c_sl = pl.ds(my * blk + c * half, half)
            for s in range(1, N_DEV):
                src = (my - s) % N_DEV
                recv_pts.append(pack_matrix(c_sl, src)[0])

        xcats = []
        for s in range(1, N_DEV):
            dest = (my + s) % N_DEV
            for c in range(N_CHUNK):
                c_sl = pl.ds(dest * blk + c * half, half)
                pt, route = pack_matrix(c_sl, my)
                xcats.append(gather_xcat(c_sl, pt, route))

        own_pre = []
        for c in range(N_CHUNK):
            c_sl = pl.ds(my * blk + c * half, half)
            gate = chunk_gate(c_sl)
            pt, route = pack_matrix(c_sl, my)
            own_pre.append((gate, pt, gather_xcat(c_sl, pt, route)))

        ew_dma.wait()
        sw_dma.wait()
        wcat16[:, :] = jnp.reshape(
            ew_vmem[:, :, :], (N_LOCAL_E * d_model, d_ff)
        ).astype(jnp.bfloat16)
        sw16[:, :] = sw_vmem[:, :].astype(jnp.bfloat16)

        rdmas = []
        for s in range(1, N_DEV):
            dest = (my + s) % N_DEV
            for c in range(N_CHUNK):
                slot = (s - 1) * N_CHUNK + c
                v = jnp.dot(xcats[slot], wcat16[:, :],
                            preferred_element_type=jnp.float32)
                q = jnp.clip(v * (1.0 / QSCALE), -127.0, 127.0)
                send_buf[slot, :, :] = jnp.round(q).astype(jnp.int8)
                rdma = pltpu.make_async_remote_copy(
                    src_ref=send_buf.at[slot],
                    dst_ref=comm_buf.at[slot],
                    send_sem=send_sems.at[slot],
                    recv_sem=recv_sems.at[slot],
                    device_id=(dest,),
                    device_id_type=pl.DeviceIdType.MESH,
                )
                rdma.start()
                rdmas.append(rdma)

        gates = []
        for c in range(N_CHUNK):
            gate, pt, xcat = own_pre[c]
            gates.append(gate)
            c_sl = pl.ds(my * blk + c * half, half)
            v = jnp.dot(xcat, wcat16[:, :], preferred_element_type=jnp.float32)
            own = jnp.dot(pt, v.astype(jnp.bfloat16),
                          preferred_element_type=jnp.float32) * gate
            xb16 = x_ref[c_sl, :].astype(jnp.bfloat16)
            own = own + jnp.dot(xb16, sw16[:, :], preferred_element_type=jnp.float32)
            out_ref[pl.ds(c * half, half), :] = own

        for c in range(N_CHUNK):
            for s in range(1, N_DEV):
                rdmas[(s - 1) * N_CHUNK + c].wait_recv()
            acc = jnp.zeros((half, d_ff), jnp.float32)
            for s in range(1, N_DEV):
                pt = recv_pts[c * (N_DEV - 1) + (s - 1)]
                q16 = comm_buf[(s - 1) * N_CHUNK + c, :, :].astype(jnp.bfloat16)
                acc = acc + jnp.dot(pt, q16, preferred_element_type=jnp.float32)
            out_sl = pl.ds(c * half, half)
            out_ref[out_sl, :] = out_ref[out_sl, :] + acc * (gates[c] * QSCALE)

        for r in rdmas:
            r.wait_send()

    return pl.pallas_call(
        body,
        out_shape=jax.ShapeDtypeStruct((blk, d_ff), jnp.float32),
        in_specs=[
            pl.BlockSpec(memory_space=pltpu.VMEM),
            pl.BlockSpec(memory_space=pltpu.VMEM),
            pl.BlockSpec(memory_space=pltpu.VMEM),
            pl.BlockSpec(memory_space=pl.ANY),
            pl.BlockSpec(memory_space=pl.ANY),
        ],
        out_specs=pl.BlockSpec(memory_space=pltpu.VMEM),
        scratch_shapes=[
            pltpu.VMEM((n_slots, CAP, d_ff), jnp.int8),
            pltpu.VMEM((n_slots, CAP, d_ff), jnp.int8),
            pltpu.VMEM((N_LOCAL_E, d_model, d_ff), jnp.float32),
            pltpu.VMEM((d_model, d_ff), jnp.float32),
            pltpu.VMEM((N_LOCAL_E * d_model, d_ff), jnp.bfloat16),
            pltpu.VMEM((d_model, d_ff), jnp.bfloat16),
            pltpu.SemaphoreType.DMA((3,)),
            pltpu.SemaphoreType.DMA((n_slots,)),
            pltpu.SemaphoreType.DMA((n_slots,)),
        ],
        compiler_params=pltpu.CompilerParams(collective_id=0),
    )(x, router_W, route_idx, expert_W, shared_W)
